# baseline (speedup 1.0000x reference)
import jax
import jax.numpy as jnp
from jax import lax
from jax.experimental import pallas as pl
from jax.experimental.pallas import tpu as pltpu
from jax.experimental.pallas import tpu_sc as plsc

NUM_CLASSES = 1000
NUM_CORES = 2
NUM_SUBCORES = 16
LANES = 16
NUM_WORKERS = NUM_CORES * NUM_SUBCORES

CBLK = 200
NBUF = 4
BCOL = 128


def _build(n_b: int, n_j: int):
    assert n_b == NUM_WORKERS * BCOL
    n_cblk = NUM_CLASSES // CBLK
    nblk = n_j * n_cblk
    full_iters = (nblk - NBUF) // NBUF
    tail = (nblk - NBUF) % NBUF
    xc_len = BCOL * n_j

    mesh = plsc.VectorSubcoreMesh(
        core_axis_name="c", subcore_axis_name="s",
        num_cores=NUM_CORES, num_subcores=NUM_SUBCORES)

    def body(x_hbm, out_hbm, xc, *scratch):
        bufs = scratch[:NBUF]
        sems = scratch[NBUF:]
        wid = lax.axis_index("s") * NUM_CORES + lax.axis_index("c")
        bcol0 = wid * BCOL

        pltpu.sync_copy(x_hbm.at[pl.ds(bcol0 * n_j, xc_len)], xc)

        zeros16 = jnp.zeros((LANES,), jnp.float32)
        ones16 = jnp.full((LANES,), 1.0, jnp.float32)
        lane_iota = lax.iota(jnp.int32, LANES)

        for b in range(NBUF):
            def zero_body(r, _, b=b):
                for k in range(BCOL // LANES):
                    bufs[b][r, pl.ds(k * LANES, LANES)] = zeros16
                return 0
            lax.fori_loop(0, CBLK, zero_body, 0)

        def scatter(b, t, val):
            j = t // n_cblk
            c0 = (t % n_cblk) * CBLK
            for g in range(BCOL // LANES):
                pos = lane_iota * n_j + (g * LANES * n_j + j)
                c = plsc.load_gather(xc, [pos])
                mask = (c >= c0) & (c < c0 + CBLK)
                plsc.store_scatter(
                    bufs[b], [c - c0, lane_iota + g * LANES], val, mask=mask)

        def start_dma(b, t):
            j = t // n_cblk
            c0 = (t % n_cblk) * CBLK
            dst = out_hbm.at[j, pl.ds(c0, CBLK), pl.ds(bcol0, BCOL)]
            pltpu.async_copy(bufs[b], dst, sems[b])

        def wait_dma(b):
            dst = out_hbm.at[0, pl.ds(0, CBLK), pl.ds(0, BCOL)]
            pltpu.make_async_copy(bufs[b], dst, sems[b]).wait()

        for b in range(NBUF):
            scatter(b, b, ones16)
            start_dma(b, b)

        def loop_body(i, carry):
            for b in range(NBUF):
                t = i * NBUF + b
                wait_dma(b)
                scatter(b, t - NBUF, zeros16)
                scatter(b, t, ones16)
                start_dma(b, t)
            return carry
        lax.fori_loop(1, full_iters + 1, loop_body, 0)

        for k in range(tail):
            t = (full_iters + 1) * NBUF + k
            b = t % NBUF
            wait_dma(b)
            scatter(b, t - NBUF, zeros16)
            scatter(b, t, ones16)
            start_dma(b, t)

        for b in range(NBUF):
            wait_dma(b)

    fn = pl.kernel(
        body,
        out_type=jax.ShapeDtypeStruct((n_j, NUM_CLASSES, n_b), jnp.float32),
        mesh=mesh,
        compiler_params=pltpu.CompilerParams(needs_layout_passes=False),
        scratch_types=(
            [pltpu.VMEM((xc_len,), jnp.int32)]
            + [pltpu.VMEM((CBLK, BCOL), jnp.float32)] * NBUF
            + [pltpu.SemaphoreType.DMA] * NBUF
        ),
    )
    return fn


def kernel(x):
    n_b, n_j = x.shape
    out = _build(n_b, n_j)(x.reshape(n_b * n_j).astype(jnp.int32))
    return jnp.transpose(out, (2, 0, 1))


# speedup vs baseline: 1.0315x; 1.0315x over previous
import jax
import jax.numpy as jnp
from jax import lax
from jax.experimental import pallas as pl
from jax.experimental.pallas import tpu as pltpu
from jax.experimental.pallas import tpu_sc as plsc

NUM_CLASSES = 1000
NUM_CORES = 2
NUM_SUBCORES = 16
LANES = 16
NUM_WORKERS = NUM_CORES * NUM_SUBCORES

CBLK = 200
NBUF = 2
BCOL = 128


def _build(n_b: int, n_j: int):
    assert n_b == NUM_WORKERS * BCOL
    n_cblk = NUM_CLASSES // CBLK
    nblk = n_j * n_cblk
    full_iters = (nblk - NBUF) // NBUF
    tail = (nblk - NBUF) % NBUF
    xc_len = BCOL * n_j

    mesh = plsc.VectorSubcoreMesh(
        core_axis_name="c", subcore_axis_name="s",
        num_cores=NUM_CORES, num_subcores=NUM_SUBCORES)

    def body(x_hbm, out_hbm, xc, *scratch):
        bufs = scratch[:NBUF]
        sems = scratch[NBUF:]
        wid = lax.axis_index("s") * NUM_CORES + lax.axis_index("c")
        bcol0 = wid * BCOL

        pltpu.sync_copy(x_hbm.at[pl.ds(bcol0 * n_j, xc_len)], xc)

        zeros16 = jnp.zeros((LANES,), jnp.float32)
        ones16 = jnp.full((LANES,), 1.0, jnp.float32)
        lane_iota = lax.iota(jnp.int32, LANES)

        for b in range(NBUF):
            def zero_body(r, _, b=b):
                for k in range(BCOL // LANES):
                    bufs[b][r, pl.ds(k * LANES, LANES)] = zeros16
                return 0
            lax.fori_loop(0, CBLK, zero_body, 0)

        def scatter(b, t, val):
            j = t // n_cblk
            c0 = (t % n_cblk) * CBLK
            for g in range(BCOL // LANES):
                pos = lane_iota * n_j + (g * LANES * n_j + j)
                c = plsc.load_gather(xc, [pos])
                mask = (c >= c0) & (c < c0 + CBLK)
                plsc.store_scatter(
                    bufs[b], [c - c0, lane_iota + g * LANES], val, mask=mask)

        def start_dma(b, t):
            j = t // n_cblk
            c0 = (t % n_cblk) * CBLK
            dst = out_hbm.at[j, pl.ds(c0, CBLK), pl.ds(bcol0, BCOL)]
            pltpu.async_copy(bufs[b], dst, sems[b])

        def wait_dma(b):
            dst = out_hbm.at[0, pl.ds(0, CBLK), pl.ds(0, BCOL)]
            pltpu.make_async_copy(bufs[b], dst, sems[b]).wait()

        for b in range(NBUF):
            scatter(b, b, ones16)
            start_dma(b, b)

        def loop_body(i, carry):
            for b in range(NBUF):
                t = i * NBUF + b
                wait_dma(b)
                scatter(b, t - NBUF, zeros16)
                scatter(b, t, ones16)
                start_dma(b, t)
            return carry
        lax.fori_loop(1, full_iters + 1, loop_body, 0)

        for k in range(tail):
            t = (full_iters + 1) * NBUF + k
            b = t % NBUF
            wait_dma(b)
            scatter(b, t - NBUF, zeros16)
            scatter(b, t, ones16)
            start_dma(b, t)

        for b in range(NBUF):
            wait_dma(b)

    fn = pl.kernel(
        body,
        out_type=jax.ShapeDtypeStruct((n_j, NUM_CLASSES, n_b), jnp.float32),
        mesh=mesh,
        compiler_params=pltpu.CompilerParams(needs_layout_passes=False),
        scratch_types=(
            [pltpu.VMEM((xc_len,), jnp.int32)]
            + [pltpu.VMEM((CBLK, BCOL), jnp.float32)] * NBUF
            + [pltpu.SemaphoreType.DMA] * NBUF
        ),
    )
    return fn


def kernel(x):
    n_b, n_j = x.shape
    out = _build(n_b, n_j)(x.reshape(n_b * n_j).astype(jnp.int32))
    return jnp.transpose(out, (2, 0, 1))
